# baseline (speedup 1.0000x reference)
import functools

import numpy as np
import jax
import jax.numpy as jnp
from jax import lax
from jax.experimental import pallas as pl
from jax.experimental.pallas import tpu as pltpu
from jax.experimental.pallas import tpu_sc as plsc

F32 = jnp.float32
NA = 10000
NP = 160000
NPP = 163840
NM = 256
NW = 32
NSEG_A = 10112
NSEG_M = 384
NAP = 10240

_MU = (-1.645, -1.080, -0.739, -0.468, -0.228, 0.0, 0.228, 0.468, 0.739,
       1.080, 1.645)
_SIG = (0.283, 0.170, 0.134, 0.118, 0.114, 0.114, 0.114, 0.118, 0.134,
        0.170, 0.283)


def _atoms_body(x_ref, w_ref, b_ref, x12_ref, aa_ref):
    y = jnp.dot(x_ref[...], w_ref[...], preferred_element_type=F32) + b_ref[...]
    x12_ref[...] = y[:, :128]
    aa_ref[...] = jnp.maximum(y[:, 128:], 0.0)


def _atoms_call(atom, wcat, bcat):
    ain = atom.shape[1]
    ba = 2000
    return pl.pallas_call(
        _atoms_body,
        grid=(NA // ba,),
        in_specs=[
            pl.BlockSpec((ba, ain), lambda i: (i, 0)),
            pl.BlockSpec((ain, 192), lambda i: (0, 0)),
            pl.BlockSpec((1, 192), lambda i: (0, 0)),
        ],
        out_specs=[
            pl.BlockSpec((ba, 128), lambda i: (i, 0)),
            pl.BlockSpec((ba, 64), lambda i: (i, 0)),
        ],
        out_shape=[
            jax.ShapeDtypeStruct((NA, 128), F32),
            jax.ShapeDtypeStruct((NA, 64), F32),
        ],
    )(atom, wcat, bcat)


def _mm_relu_body(x_ref, w_ref, b_ref, o_ref):
    o_ref[...] = jnp.maximum(
        jnp.dot(x_ref[...], w_ref[...], preferred_element_type=F32)
        + b_ref[...], 0.0)


def _pa_call(pair, w, b):
    pin = pair.shape[1]
    bp = 4096
    return pl.pallas_call(
        _mm_relu_body,
        grid=(NPP // bp,),
        in_specs=[
            pl.BlockSpec((bp, pin), lambda i: (i, 0)),
            pl.BlockSpec((pin, 64), lambda i: (0, 0)),
            pl.BlockSpec((1, 64), lambda i: (0, 0)),
        ],
        out_specs=pl.BlockSpec((bp, 64), lambda i: (i, 0)),
        out_shape=jax.ShapeDtypeStruct((NPP, 64), F32),
    )(pair, w, b)


def _aa_call(atom, w, b):
    ain = atom.shape[1]
    ba = 2000
    return pl.pallas_call(
        _mm_relu_body,
        grid=(NA // ba,),
        in_specs=[
            pl.BlockSpec((ba, ain), lambda i: (i, 0)),
            pl.BlockSpec((ain, 64), lambda i: (0, 0)),
            pl.BlockSpec((1, 64), lambda i: (0, 0)),
        ],
        out_specs=pl.BlockSpec((ba, 64), lambda i: (i, 0)),
        out_shape=jax.ShapeDtypeStruct((NA, 64), F32),
    )(atom, w, b)


def _A_body(aa_ref, s0_ref, s1_ref, w1_ref, w2_ref, b_ref, o_ref):
    y = jnp.dot(aa_ref[...], w1_ref[...], preferred_element_type=F32)
    y += jnp.dot(s0_ref[...] + s1_ref[...], w2_ref[...],
                 preferred_element_type=F32)
    o_ref[...] = jnp.maximum(y + b_ref[...], 0.0)


def _A_call(aa, s0, s1, w1, w2, b):
    ba = 2000
    return pl.pallas_call(
        _A_body,
        grid=(NA // ba,),
        in_specs=[
            pl.BlockSpec((ba, 64), lambda i: (i, 0)),
            pl.BlockSpec((ba, 64), lambda i: (i, 0)),
            pl.BlockSpec((ba, 64), lambda i: (i, 0)),
            pl.BlockSpec((64, 64), lambda i: (0, 0)),
            pl.BlockSpec((64, 64), lambda i: (0, 0)),
            pl.BlockSpec((1, 64), lambda i: (0, 0)),
        ],
        out_specs=pl.BlockSpec((ba, 64), lambda i: (i, 0)),
        out_shape=jax.ShapeDtypeStruct((NA, 64), F32),
    )(aa, s0, s1, w1, w2, b)


def _P_fuse_body(ap_ref, pr_ref, wp1_ref, wp2_ref, wpp_ref, bpp_ref, bp_ref,
                 wpa_ref, bpa_ref, o_ref):
    pp = jnp.maximum(
        jnp.dot(pr_ref[...], wpp_ref[...], preferred_element_type=F32)
        + bpp_ref[...], 0.0)
    y = jnp.dot(ap_ref[...], wp1_ref[...], preferred_element_type=F32)
    y += jnp.dot(pp, wp2_ref[...], preferred_element_type=F32)
    p = jnp.maximum(y + bp_ref[...], 0.0)
    o_ref[...] = jnp.maximum(
        jnp.dot(p, wpa_ref[...], preferred_element_type=F32) + bpa_ref[...],
        0.0)


def _P_fuse_call(ap, pair, wp1, wp2, wpp, bpp, bp, wpa, bpa):
    pin = pair.shape[1]
    bpr = 4096
    return pl.pallas_call(
        _P_fuse_body,
        grid=(NPP // bpr,),
        in_specs=[
            pl.BlockSpec((bpr, 64), lambda i: (i, 0)),
            pl.BlockSpec((bpr, pin), lambda i: (i, 0)),
            pl.BlockSpec((64, 64), lambda i: (0, 0)),
            pl.BlockSpec((64, 64), lambda i: (0, 0)),
            pl.BlockSpec((pin, 64), lambda i: (0, 0)),
            pl.BlockSpec((1, 64), lambda i: (0, 0)),
            pl.BlockSpec((1, 64), lambda i: (0, 0)),
            pl.BlockSpec((64, 64), lambda i: (0, 0)),
            pl.BlockSpec((1, 64), lambda i: (0, 0)),
        ],
        out_specs=pl.BlockSpec((bpr, 64), lambda i: (i, 0)),
        out_shape=jax.ShapeDtypeStruct((NPP, 64), F32),
    )(ap, pair, wp1, wp2, wpp, bpp, bp, wpa, bpa)


def _head_body(a_ref, wd_ref, bd_ref, sc_ref, be_ref, wg_ref, o_ref, m_ref):
    h = jnp.tanh(
        jnp.dot(a_ref[...], wd_ref[...], preferred_element_type=F32)
        + bd_ref[...])
    h = h * sc_ref[...] + be_ref[...]
    den = jnp.zeros_like(h)
    for k in range(11):
        c = -0.5 / (_SIG[k] * _SIG[k])
        d = h - _MU[k]
        mk = jnp.exp(c * d * d)
        m_ref[k] = mk
        den = den + mk
    inv = 1.0 / den
    acc = jnp.zeros(o_ref.shape, F32)
    for k in range(11):
        acc = acc + jnp.dot(m_ref[k] * inv, wg_ref[k * 128:(k + 1) * 128, :],
                            preferred_element_type=F32)
    o_ref[...] = acc


def _head_call(a, wd, bd, scale, beta, wg):
    bh = 512
    return pl.pallas_call(
        _head_body,
        grid=(NAP // bh,),
        in_specs=[
            pl.BlockSpec((bh, 64), lambda i: (i, 0)),
            pl.BlockSpec((64, 128), lambda i: (0, 0)),
            pl.BlockSpec((1, 128), lambda i: (0, 0)),
            pl.BlockSpec((1, 128), lambda i: (0, 0)),
            pl.BlockSpec((1, 128), lambda i: (0, 0)),
            pl.BlockSpec((1408, 128), lambda i: (0, 0)),
        ],
        out_specs=pl.BlockSpec((bh, 128), lambda i: (i, 0)),
        out_shape=jax.ShapeDtypeStruct((NAP, 128), F32),
        scratch_shapes=[pltpu.VMEM((11, bh, 128), F32)],
    )(a, wd, bd, scale, beta, wg)


def _out_body(p0_ref, p1_ref, b_ref, o_ref):
    o_ref[...] = jnp.tanh(p0_ref[...] + p1_ref[...] + b_ref[...])


def _out_call(p0, p1, b):
    return pl.pallas_call(
        _out_body,
        grid=(1,),
        in_specs=[
            pl.BlockSpec((NM, 128), lambda i: (0, 0)),
            pl.BlockSpec((NM, 128), lambda i: (0, 0)),
            pl.BlockSpec((1, 128), lambda i: (0, 0)),
        ],
        out_specs=pl.BlockSpec((NM, 128), lambda i: (0, 0)),
        out_shape=jax.ShapeDtypeStruct((NM, 128), F32),
    )(p0, p1, b)


_MESH = plsc.VectorSubcoreMesh(core_axis_name="c", subcore_axis_name="s")


def _make_segsum(nrows, d, nseg, g, steps):
    rw = nrows // NW
    sr = rw // steps
    gs = sr // g
    gpw = gs * steps
    zr = nseg // 16
    assert sr % 8 == 0 and gpw % 8 == 0 and zr % 8 == 0 and gs * g == sr

    def body(vals_hbm, idx_hbm, zeros_hbm, out_hbm, acc_sh,
             vals0, vals1, idx_v, sem0, sem1):
        c = lax.axis_index("c")
        s = lax.axis_index("s")
        wid = s * 2 + c
        pltpu.sync_copy(zeros_hbm, acc_sh.at[pl.ds(s * zr, zr)])
        pltpu.sync_copy(idx_hbm.at[pl.ds(wid * gpw, gpw)], idx_v)
        plsc.subcore_barrier()

        bufs = ((vals0, sem0), (vals1, sem1))
        descs = [None, None]

        def fire(st):
            b = st % 2
            vals_v, sem = bufs[b]
            descs[b] = pltpu.async_copy(
                vals_hbm.at[pl.ds(wid * rw + st * sr, sr)], vals_v, sem)

        fire(0)
        for st in range(steps):
            b = st % 2
            if st + 1 < steps:
                fire(st + 1)
            descs[b].wait()
            vals_v, _ = bufs[b]
            for j in range(gs):
                pltpu.sync_copy(vals_v.at[pl.ds(j * g, g)],
                                acc_sh.at[idx_v.at[st * gs + j]], add=True)
        plsc.subcore_barrier()
        pltpu.sync_copy(acc_sh.at[pl.ds(s * zr, zr)],
                        out_hbm.at[pl.ds(c * nseg + s * zr, zr)])

    return pl.kernel(
        body,
        out_type=jax.ShapeDtypeStruct((2 * nseg, d), F32),
        mesh=_MESH,
        scratch_types=[
            pltpu.VMEM_SHARED((nseg, d), F32),
            pltpu.VMEM((sr, d), F32),
            pltpu.VMEM((sr, d), F32),
            pltpu.VMEM((gpw, g), jnp.int32),
            pltpu.SemaphoreType.DMA,
            pltpu.SemaphoreType.DMA,
        ])


def _make_gather():
    g = 128
    GT = NPP // NW // g * 2
    G_SLOW = 24
    G_FAST = GT - G_SLOW
    GMAX = max(G_SLOW, G_FAST)

    def body(x12_hbm, ii_hbm, jj_hbm, ap_hbm,
             gi0, gi1, gj0, gj1, ov0, ov1, ii_v, jj_v,
             semg0, semg1, semo0, semo1):
        c = lax.axis_index("c")
        s = lax.axis_index("s")
        base_g = s * GT + jnp.where(c == 0, 0, G_SLOW)
        ngrp = jnp.where(c == 0, G_SLOW, G_FAST)
        pltpu.sync_copy(ii_hbm.at[pl.ds(base_g, GMAX)], ii_v)
        pltpu.sync_copy(jj_hbm.at[pl.ds(base_g, GMAX)], jj_v)

        gbufs = ((gi0, gj0, semg0), (gi1, gj1, semg1))
        obufs = ((ov0, semo0), (ov1, semo1))
        base = base_g * g

        def fire(grp, b):
            gi, gj, sem = gbufs[b]
            pltpu.async_copy(x12_hbm.at[ii_v.at[grp]], gi, sem)
            pltpu.async_copy(x12_hbm.at[jj_v.at[grp]], gj, sem)

        def wait_gather(b):
            gi, gj, sem = gbufs[b]
            pltpu.make_async_copy(x12_hbm.at[ii_v.at[0]], gi, sem).wait()
            pltpu.make_async_copy(x12_hbm.at[jj_v.at[0]], gj, sem).wait()

        def wait_out(b):
            ov, semo = obufs[b]
            pltpu.make_async_copy(ov, ap_hbm.at[pl.ds(0, g)], semo).wait()

        def process(grp, b, st):
            gi, gj, _ = gbufs[b]
            ov, semo = obufs[b]
            wait_gather(b)

            @pl.when(st > 0)
            def _():
                wait_out(b)

            def row(r, carry):
                for q in range(4):
                    x1i = gi[r, pl.ds(q * 16, 16)]
                    x2i = gi[r, pl.ds(64 + q * 16, 16)]
                    x1j = gj[r, pl.ds(q * 16, 16)]
                    x2j = gj[r, pl.ds(64 + q * 16, 16)]
                    t = (jnp.maximum(x1i + x2j, 0.0)
                         + jnp.maximum(x1j + x2i, 0.0))
                    ov[r, pl.ds(q * 16, 16)] = t
                return carry

            lax.fori_loop(0, g, row, 0, unroll=4)
            pltpu.async_copy(ov, ap_hbm.at[pl.ds(base + grp * g, g)], semo)

        nsteps = ngrp // 2
        fire(0, 0)

        def step(st, carry):
            g0 = st * 2
            fire(g0 + 1, 1)
            process(g0, 0, st)

            @pl.when(st < nsteps - 1)
            def _():
                fire(g0 + 2, 0)

            process(g0 + 1, 1, st)
            return carry

        lax.fori_loop(0, nsteps, step, 0)
        wait_out(0)
        wait_out(1)

    return pl.kernel(
        body,
        out_type=jax.ShapeDtypeStruct((NPP, 64), F32),
        mesh=_MESH,
        scratch_types=[
            pltpu.VMEM((g, 128), F32),
            pltpu.VMEM((g, 128), F32),
            pltpu.VMEM((g, 128), F32),
            pltpu.VMEM((g, 128), F32),
            pltpu.VMEM((g, 64), F32),
            pltpu.VMEM((g, 64), F32),
            pltpu.VMEM((GMAX, g), jnp.int32),
            pltpu.VMEM((GMAX, g), jnp.int32),
            pltpu.SemaphoreType.DMA,
            pltpu.SemaphoreType.DMA,
            pltpu.SemaphoreType.DMA,
            pltpu.SemaphoreType.DMA,
        ])


def kernel(atom_features, pair_features, params, pair_split, atom_split,
           atom_to_pair):
    pad_p = NPP - NP
    ps2 = jnp.concatenate([
        pair_split.astype(jnp.int32),
        jnp.full((pad_p,), NA, jnp.int32),
    ]).reshape(NPP // 128, 128)
    ii2 = jnp.concatenate([
        atom_to_pair[:, 0].astype(jnp.int32), jnp.zeros((pad_p,), jnp.int32),
    ]).reshape(NPP // 128, 128)
    jj2 = jnp.concatenate([
        atom_to_pair[:, 1].astype(jnp.int32), jnp.zeros((pad_p,), jnp.int32),
    ]).reshape(NPP // 128, 128)
    asp = jnp.concatenate([
        atom_split.astype(jnp.int32),
        jnp.full((NAP - NA,), NM, jnp.int32),
    ]).reshape(NAP // 40, 40)
    zeros_a = jnp.zeros((NSEG_A // 16, 64), F32)
    zeros_m = jnp.zeros((NSEG_M // 16, 128), F32)

    segsum_pairs = _make_segsum(NPP, 64, NSEG_A, g=128, steps=40)
    segsum_mols = _make_segsum(NAP, 128, NSEG_M, g=40, steps=1)
    gather_pairs = _make_gather()

    w_ap = params["W_AP0"]
    wcat = jnp.concatenate([w_ap[:128], w_ap[128:], params["W_AA0"]], axis=1)
    bcat = jnp.concatenate([
        jnp.zeros((64,), F32), params["b_AP0"],
        params["b_AA0"]]).reshape(1, 192)
    x12, aa0 = _atoms_call(atom_features, wcat, bcat)
    pa0 = _pa_call(pair_features, params["W_PA0"],
                   params["b_PA0"].reshape(1, 64))
    seg0 = segsum_pairs(pa0, ps2, zeros_a)
    wa = params["W_A0"]
    atom1 = _A_call(aa0, seg0[:NA], seg0[NSEG_A:NSEG_A + NA], wa[:64],
                    wa[64:], params["b_A0"].reshape(1, 64))
    ap0 = gather_pairs(x12, ii2, jj2)
    wp = params["W_P0"]
    pa1 = _P_fuse_call(ap0, pair_features, wp[:64], wp[64:], params["W_PP0"],
                       params["b_PP0"].reshape(1, 64),
                       params["b_P0"].reshape(1, 64),
                       params["W_PA1"], params["b_PA1"].reshape(1, 64))
    aa1 = _aa_call(atom1, params["W_AA1"], params["b_AA1"].reshape(1, 64))
    seg1 = segsum_pairs(pa1, ps2, zeros_a)
    wa = params["W_A1"]
    atom = _A_call(aa1, seg1[:NA], seg1[NSEG_A:NSEG_A + NA], wa[:64],
                   wa[64:], params["b_A1"].reshape(1, 64))

    scale = (params["bn_gamma"] / np.sqrt(1.0 + 1e-3)).reshape(1, 128)
    beta = params["bn_beta"].reshape(1, 128)
    wg = jnp.transpose(params["W_g"].reshape(128, 11, 128),
                       (1, 0, 2)).reshape(1408, 128)
    gpre = _head_call(atom, params["W_d"], params["b_d"].reshape(1, 128),
                      scale, beta, wg)
    msum = segsum_mols(gpre, asp, zeros_m)
    return _out_call(msum[:NM], msum[NSEG_M:NSEG_M + NM],
                     params["b_g"].reshape(1, 128))


# speedup vs baseline: 1.0007x; 1.0007x over previous
import functools

import numpy as np
import jax
import jax.numpy as jnp
from jax import lax
from jax.experimental import pallas as pl
from jax.experimental.pallas import tpu as pltpu
from jax.experimental.pallas import tpu_sc as plsc

F32 = jnp.float32
NA = 10000
NP = 160000
NPP = 163840
NM = 256
NW = 32
NSEG_A = 10112
NSEG_M = 384
NAP = 10240

_MU = (-1.645, -1.080, -0.739, -0.468, -0.228, 0.0, 0.228, 0.468, 0.739,
       1.080, 1.645)
_SIG = (0.283, 0.170, 0.134, 0.118, 0.114, 0.114, 0.114, 0.118, 0.134,
        0.170, 0.283)


def _atoms_body(x_ref, w_ref, b_ref, x12_ref, aa_ref):
    y = jnp.dot(x_ref[...], w_ref[...], preferred_element_type=F32) + b_ref[...]
    x12_ref[...] = y[:, :128]
    aa_ref[...] = jnp.maximum(y[:, 128:], 0.0)


def _atoms_call(atom, wcat, bcat):
    ain = atom.shape[1]
    ba = 2000
    return pl.pallas_call(
        _atoms_body,
        grid=(NA // ba,),
        in_specs=[
            pl.BlockSpec((ba, ain), lambda i: (i, 0)),
            pl.BlockSpec((ain, 192), lambda i: (0, 0)),
            pl.BlockSpec((1, 192), lambda i: (0, 0)),
        ],
        out_specs=[
            pl.BlockSpec((ba, 128), lambda i: (i, 0)),
            pl.BlockSpec((ba, 64), lambda i: (i, 0)),
        ],
        out_shape=[
            jax.ShapeDtypeStruct((NA, 128), F32),
            jax.ShapeDtypeStruct((NA, 64), F32),
        ],
    )(atom, wcat, bcat)


def _mm_relu_body(x_ref, w_ref, b_ref, o_ref):
    o_ref[...] = jnp.maximum(
        jnp.dot(x_ref[...], w_ref[...], preferred_element_type=F32)
        + b_ref[...], 0.0)


def _pa_call(pair, w, b):
    pin = pair.shape[1]
    bp = 4096
    return pl.pallas_call(
        _mm_relu_body,
        grid=(NPP // bp,),
        in_specs=[
            pl.BlockSpec((bp, pin), lambda i: (i, 0)),
            pl.BlockSpec((pin, 64), lambda i: (0, 0)),
            pl.BlockSpec((1, 64), lambda i: (0, 0)),
        ],
        out_specs=pl.BlockSpec((bp, 64), lambda i: (i, 0)),
        out_shape=jax.ShapeDtypeStruct((NPP, 64), F32),
    )(pair, w, b)


def _aa_call(atom, w, b):
    ain = atom.shape[1]
    ba = 2000
    return pl.pallas_call(
        _mm_relu_body,
        grid=(NA // ba,),
        in_specs=[
            pl.BlockSpec((ba, ain), lambda i: (i, 0)),
            pl.BlockSpec((ain, 64), lambda i: (0, 0)),
            pl.BlockSpec((1, 64), lambda i: (0, 0)),
        ],
        out_specs=pl.BlockSpec((ba, 64), lambda i: (i, 0)),
        out_shape=jax.ShapeDtypeStruct((NA, 64), F32),
    )(atom, w, b)


def _A_body(aa_ref, s0_ref, s1_ref, w1_ref, w2_ref, b_ref, o_ref):
    y = jnp.dot(aa_ref[...], w1_ref[...], preferred_element_type=F32)
    y += jnp.dot(s0_ref[...] + s1_ref[...], w2_ref[...],
                 preferred_element_type=F32)
    o_ref[...] = jnp.maximum(y + b_ref[...], 0.0)


def _A_call(aa, s0, s1, w1, w2, b):
    ba = 2000
    return pl.pallas_call(
        _A_body,
        grid=(NA // ba,),
        in_specs=[
            pl.BlockSpec((ba, 64), lambda i: (i, 0)),
            pl.BlockSpec((ba, 64), lambda i: (i, 0)),
            pl.BlockSpec((ba, 64), lambda i: (i, 0)),
            pl.BlockSpec((64, 64), lambda i: (0, 0)),
            pl.BlockSpec((64, 64), lambda i: (0, 0)),
            pl.BlockSpec((1, 64), lambda i: (0, 0)),
        ],
        out_specs=pl.BlockSpec((ba, 64), lambda i: (i, 0)),
        out_shape=jax.ShapeDtypeStruct((NA, 64), F32),
    )(aa, s0, s1, w1, w2, b)


def _P_fuse_body(ap_ref, pr_ref, wp1_ref, wp2_ref, wpp_ref, bpp_ref, bp_ref,
                 wpa_ref, bpa_ref, o_ref):
    pp = jnp.maximum(
        jnp.dot(pr_ref[...], wpp_ref[...], preferred_element_type=F32)
        + bpp_ref[...], 0.0)
    y = jnp.dot(ap_ref[...], wp1_ref[...], preferred_element_type=F32)
    y += jnp.dot(pp, wp2_ref[...], preferred_element_type=F32)
    p = jnp.maximum(y + bp_ref[...], 0.0)
    o_ref[...] = jnp.maximum(
        jnp.dot(p, wpa_ref[...], preferred_element_type=F32) + bpa_ref[...],
        0.0)


def _P_fuse_call(ap, pair, wp1, wp2, wpp, bpp, bp, wpa, bpa):
    pin = pair.shape[1]
    bpr = 4096
    return pl.pallas_call(
        _P_fuse_body,
        grid=(NPP // bpr,),
        in_specs=[
            pl.BlockSpec((bpr, 64), lambda i: (i, 0)),
            pl.BlockSpec((bpr, pin), lambda i: (i, 0)),
            pl.BlockSpec((64, 64), lambda i: (0, 0)),
            pl.BlockSpec((64, 64), lambda i: (0, 0)),
            pl.BlockSpec((pin, 64), lambda i: (0, 0)),
            pl.BlockSpec((1, 64), lambda i: (0, 0)),
            pl.BlockSpec((1, 64), lambda i: (0, 0)),
            pl.BlockSpec((64, 64), lambda i: (0, 0)),
            pl.BlockSpec((1, 64), lambda i: (0, 0)),
        ],
        out_specs=pl.BlockSpec((bpr, 64), lambda i: (i, 0)),
        out_shape=jax.ShapeDtypeStruct((NPP, 64), F32),
    )(ap, pair, wp1, wp2, wpp, bpp, bp, wpa, bpa)


def _head_body(a_ref, wd_ref, bd_ref, sc_ref, be_ref, wg_ref, o_ref, m_ref):
    h = jnp.tanh(
        jnp.dot(a_ref[...], wd_ref[...], preferred_element_type=F32)
        + bd_ref[...])
    h = h * sc_ref[...] + be_ref[...]
    den = jnp.zeros_like(h)
    for k in range(11):
        c = -0.5 / (_SIG[k] * _SIG[k])
        d = h - _MU[k]
        mk = jnp.exp(c * d * d)
        m_ref[k] = mk
        den = den + mk
    inv = 1.0 / den
    acc = jnp.zeros(o_ref.shape, F32)
    for k in range(11):
        acc = acc + jnp.dot(m_ref[k] * inv, wg_ref[k * 128:(k + 1) * 128, :],
                            preferred_element_type=F32)
    o_ref[...] = acc


def _head_call(a, wd, bd, scale, beta, wg):
    bh = 512
    return pl.pallas_call(
        _head_body,
        grid=(NAP // bh,),
        in_specs=[
            pl.BlockSpec((bh, 64), lambda i: (i, 0)),
            pl.BlockSpec((64, 128), lambda i: (0, 0)),
            pl.BlockSpec((1, 128), lambda i: (0, 0)),
            pl.BlockSpec((1, 128), lambda i: (0, 0)),
            pl.BlockSpec((1, 128), lambda i: (0, 0)),
            pl.BlockSpec((1408, 128), lambda i: (0, 0)),
        ],
        out_specs=pl.BlockSpec((bh, 128), lambda i: (i, 0)),
        out_shape=jax.ShapeDtypeStruct((NAP, 128), F32),
        scratch_shapes=[pltpu.VMEM((11, bh, 128), F32)],
    )(a, wd, bd, scale, beta, wg)


def _out_body(p0_ref, p1_ref, b_ref, o_ref):
    o_ref[...] = jnp.tanh(p0_ref[...] + p1_ref[...] + b_ref[...])


def _out_call(p0, p1, b):
    return pl.pallas_call(
        _out_body,
        grid=(1,),
        in_specs=[
            pl.BlockSpec((NM, 128), lambda i: (0, 0)),
            pl.BlockSpec((NM, 128), lambda i: (0, 0)),
            pl.BlockSpec((1, 128), lambda i: (0, 0)),
        ],
        out_specs=pl.BlockSpec((NM, 128), lambda i: (0, 0)),
        out_shape=jax.ShapeDtypeStruct((NM, 128), F32),
    )(p0, p1, b)


_MESH = plsc.VectorSubcoreMesh(core_axis_name="c", subcore_axis_name="s")


def _make_segsum(nrows, d, nseg, g, steps):
    rw = nrows // NW
    sr = rw // steps
    gs = sr // g
    gpw = gs * steps
    zr = nseg // 16
    assert sr % 8 == 0 and gpw % 8 == 0 and zr % 8 == 0 and gs * g == sr

    def body(vals_hbm, idx_hbm, zeros_hbm, out_hbm, acc_sh,
             vals0, vals1, idx_v, sem0, sem1):
        c = lax.axis_index("c")
        s = lax.axis_index("s")
        wid = s * 2 + c
        pltpu.sync_copy(zeros_hbm, acc_sh.at[pl.ds(s * zr, zr)])
        pltpu.sync_copy(idx_hbm.at[pl.ds(wid * gpw, gpw)], idx_v)
        plsc.subcore_barrier()

        bufs = ((vals0, sem0), (vals1, sem1))
        descs = [None, None]

        def fire(st):
            b = st % 2
            vals_v, sem = bufs[b]
            descs[b] = pltpu.async_copy(
                vals_hbm.at[pl.ds(wid * rw + st * sr, sr)], vals_v, sem)

        fire(0)
        for st in range(steps):
            b = st % 2
            if st + 1 < steps:
                fire(st + 1)
            descs[b].wait()
            vals_v, _ = bufs[b]
            for j in range(gs):
                pltpu.sync_copy(vals_v.at[pl.ds(j * g, g)],
                                acc_sh.at[idx_v.at[st * gs + j]], add=True)
        plsc.subcore_barrier()
        pltpu.sync_copy(acc_sh.at[pl.ds(s * zr, zr)],
                        out_hbm.at[pl.ds(c * nseg + s * zr, zr)])

    return pl.kernel(
        body,
        out_type=jax.ShapeDtypeStruct((2 * nseg, d), F32),
        mesh=_MESH,
        scratch_types=[
            pltpu.VMEM_SHARED((nseg, d), F32),
            pltpu.VMEM((sr, d), F32),
            pltpu.VMEM((sr, d), F32),
            pltpu.VMEM((gpw, g), jnp.int32),
            pltpu.SemaphoreType.DMA,
            pltpu.SemaphoreType.DMA,
        ])


def _make_gather():
    g = 128
    GT = NPP // NW // g * 2
    G_C0 = 56
    G_C1 = GT - G_C0
    GMAX = max(G_C0, G_C1)

    def body(x12_hbm, ii_hbm, jj_hbm, ap_hbm,
             gi0, gi1, gj0, gj1, ov0, ov1, ii_v, jj_v,
             semg0, semg1, semo0, semo1):
        c = lax.axis_index("c")
        s = lax.axis_index("s")
        base_g = s * GT + jnp.where(c == 0, 0, G_C0)
        ngrp = jnp.where(c == 0, G_C0, G_C1)
        pltpu.sync_copy(ii_hbm.at[pl.ds(base_g, GMAX)], ii_v)
        pltpu.sync_copy(jj_hbm.at[pl.ds(base_g, GMAX)], jj_v)

        gbufs = ((gi0, gj0, semg0), (gi1, gj1, semg1))
        obufs = ((ov0, semo0), (ov1, semo1))
        base = base_g * g

        def fire(grp, b):
            gi, gj, sem = gbufs[b]
            pltpu.async_copy(x12_hbm.at[ii_v.at[grp]], gi, sem)
            pltpu.async_copy(x12_hbm.at[jj_v.at[grp]], gj, sem)

        def wait_gather(b):
            gi, gj, sem = gbufs[b]
            pltpu.make_async_copy(x12_hbm.at[ii_v.at[0]], gi, sem).wait()
            pltpu.make_async_copy(x12_hbm.at[jj_v.at[0]], gj, sem).wait()

        def wait_out(b):
            ov, semo = obufs[b]
            pltpu.make_async_copy(ov, ap_hbm.at[pl.ds(0, g)], semo).wait()

        def process(grp, b, st):
            gi, gj, _ = gbufs[b]
            ov, semo = obufs[b]
            wait_gather(b)

            @pl.when(st > 0)
            def _():
                wait_out(b)

            def row(r, carry):
                for q in range(4):
                    x1i = gi[r, pl.ds(q * 16, 16)]
                    x2i = gi[r, pl.ds(64 + q * 16, 16)]
                    x1j = gj[r, pl.ds(q * 16, 16)]
                    x2j = gj[r, pl.ds(64 + q * 16, 16)]
                    t = (jnp.maximum(x1i + x2j, 0.0)
                         + jnp.maximum(x1j + x2i, 0.0))
                    ov[r, pl.ds(q * 16, 16)] = t
                return carry

            lax.fori_loop(0, g, row, 0, unroll=4)
            pltpu.async_copy(ov, ap_hbm.at[pl.ds(base + grp * g, g)], semo)

        nsteps = ngrp // 2
        fire(0, 0)

        def step(st, carry):
            g0 = st * 2
            fire(g0 + 1, 1)
            process(g0, 0, st)

            @pl.when(st < nsteps - 1)
            def _():
                fire(g0 + 2, 0)

            process(g0 + 1, 1, st)
            return carry

        lax.fori_loop(0, nsteps, step, 0)
        wait_out(0)
        wait_out(1)

    return pl.kernel(
        body,
        out_type=jax.ShapeDtypeStruct((NPP, 64), F32),
        mesh=_MESH,
        scratch_types=[
            pltpu.VMEM((g, 128), F32),
            pltpu.VMEM((g, 128), F32),
            pltpu.VMEM((g, 128), F32),
            pltpu.VMEM((g, 128), F32),
            pltpu.VMEM((g, 64), F32),
            pltpu.VMEM((g, 64), F32),
            pltpu.VMEM((GMAX, g), jnp.int32),
            pltpu.VMEM((GMAX, g), jnp.int32),
            pltpu.SemaphoreType.DMA,
            pltpu.SemaphoreType.DMA,
            pltpu.SemaphoreType.DMA,
            pltpu.SemaphoreType.DMA,
        ])


def kernel(atom_features, pair_features, params, pair_split, atom_split,
           atom_to_pair):
    pad_p = NPP - NP
    ps2 = jnp.concatenate([
        pair_split.astype(jnp.int32),
        jnp.full((pad_p,), NA, jnp.int32),
    ]).reshape(NPP // 128, 128)
    ii2 = jnp.concatenate([
        atom_to_pair[:, 0].astype(jnp.int32), jnp.zeros((pad_p,), jnp.int32),
    ]).reshape(NPP // 128, 128)
    jj2 = jnp.concatenate([
        atom_to_pair[:, 1].astype(jnp.int32), jnp.zeros((pad_p,), jnp.int32),
    ]).reshape(NPP // 128, 128)
    asp = jnp.concatenate([
        atom_split.astype(jnp.int32),
        jnp.full((NAP - NA,), NM, jnp.int32),
    ]).reshape(NAP // 40, 40)
    zeros_a = jnp.zeros((NSEG_A // 16, 64), F32)
    zeros_m = jnp.zeros((NSEG_M // 16, 128), F32)

    segsum_pairs = _make_segsum(NPP, 64, NSEG_A, g=128, steps=40)
    segsum_mols = _make_segsum(NAP, 128, NSEG_M, g=40, steps=1)
    gather_pairs = _make_gather()

    w_ap = params["W_AP0"]
    wcat = jnp.concatenate([w_ap[:128], w_ap[128:], params["W_AA0"]], axis=1)
    bcat = jnp.concatenate([
        jnp.zeros((64,), F32), params["b_AP0"],
        params["b_AA0"]]).reshape(1, 192)
    x12, aa0 = _atoms_call(atom_features, wcat, bcat)
    pa0 = _pa_call(pair_features, params["W_PA0"],
                   params["b_PA0"].reshape(1, 64))
    seg0 = segsum_pairs(pa0, ps2, zeros_a)
    wa = params["W_A0"]
    atom1 = _A_call(aa0, seg0[:NA], seg0[NSEG_A:NSEG_A + NA], wa[:64],
                    wa[64:], params["b_A0"].reshape(1, 64))
    ap0 = gather_pairs(x12, ii2, jj2)
    wp = params["W_P0"]
    pa1 = _P_fuse_call(ap0, pair_features, wp[:64], wp[64:], params["W_PP0"],
                       params["b_PP0"].reshape(1, 64),
                       params["b_P0"].reshape(1, 64),
                       params["W_PA1"], params["b_PA1"].reshape(1, 64))
    aa1 = _aa_call(atom1, params["W_AA1"], params["b_AA1"].reshape(1, 64))
    seg1 = segsum_pairs(pa1, ps2, zeros_a)
    wa = params["W_A1"]
    atom = _A_call(aa1, seg1[:NA], seg1[NSEG_A:NSEG_A + NA], wa[:64],
                   wa[64:], params["b_A1"].reshape(1, 64))

    scale = (params["bn_gamma"] / np.sqrt(1.0 + 1e-3)).reshape(1, 128)
    beta = params["bn_beta"].reshape(1, 128)
    wg = jnp.transpose(params["W_g"].reshape(128, 11, 128),
                       (1, 0, 2)).reshape(1408, 128)
    gpre = _head_call(atom, params["W_d"], params["b_d"].reshape(1, 128),
                      scale, beta, wg)
    msum = segsum_mols(gpre, asp, zeros_m)
    return _out_call(msum[:NM], msum[NSEG_M:NSEG_M + NM],
                     params["b_g"].reshape(1, 128))


# speedup vs baseline: 1.0282x; 1.0274x over previous
import functools

import numpy as np
import jax
import jax.numpy as jnp
from jax import lax
from jax.experimental import pallas as pl
from jax.experimental.pallas import tpu as pltpu
from jax.experimental.pallas import tpu_sc as plsc

F32 = jnp.float32
NA = 10000
NP = 160000
NPP = 163840
NM = 256
NW = 32
NSEG_A = 10112
NSEG_M = 384
NAP = 10240

_MU = (-1.645, -1.080, -0.739, -0.468, -0.228, 0.0, 0.228, 0.468, 0.739,
       1.080, 1.645)
_SIG = (0.283, 0.170, 0.134, 0.118, 0.114, 0.114, 0.114, 0.118, 0.134,
        0.170, 0.283)


def _atoms_body(x_ref, w_ref, b_ref, x12_ref, aa_ref):
    y = jnp.dot(x_ref[...], w_ref[...], preferred_element_type=F32) + b_ref[...]
    x12_ref[...] = y[:, :128]
    aa_ref[...] = jnp.maximum(y[:, 128:], 0.0)


def _atoms_call(atom, wcat, bcat):
    ain = atom.shape[1]
    ba = 2000
    return pl.pallas_call(
        _atoms_body,
        grid=(NA // ba,),
        in_specs=[
            pl.BlockSpec((ba, ain), lambda i: (i, 0)),
            pl.BlockSpec((ain, 192), lambda i: (0, 0)),
            pl.BlockSpec((1, 192), lambda i: (0, 0)),
        ],
        out_specs=[
            pl.BlockSpec((ba, 128), lambda i: (i, 0)),
            pl.BlockSpec((ba, 64), lambda i: (i, 0)),
        ],
        out_shape=[
            jax.ShapeDtypeStruct((NA, 128), F32),
            jax.ShapeDtypeStruct((NA, 64), F32),
        ],
    )(atom, wcat, bcat)


def _mm_relu_body(x_ref, w_ref, b_ref, o_ref):
    o_ref[...] = jnp.maximum(
        jnp.dot(x_ref[...], w_ref[...], preferred_element_type=F32)
        + b_ref[...], 0.0)


def _pa_call(pair, w, b):
    pin = pair.shape[1]
    bp = 4096
    return pl.pallas_call(
        _mm_relu_body,
        grid=(NPP // bp,),
        in_specs=[
            pl.BlockSpec((bp, pin), lambda i: (i, 0)),
            pl.BlockSpec((pin, 64), lambda i: (0, 0)),
            pl.BlockSpec((1, 64), lambda i: (0, 0)),
        ],
        out_specs=pl.BlockSpec((bp, 64), lambda i: (i, 0)),
        out_shape=jax.ShapeDtypeStruct((NPP, 64), F32),
    )(pair, w, b)


def _aa_call(atom, w, b):
    ain = atom.shape[1]
    ba = 2000
    return pl.pallas_call(
        _mm_relu_body,
        grid=(NA // ba,),
        in_specs=[
            pl.BlockSpec((ba, ain), lambda i: (i, 0)),
            pl.BlockSpec((ain, 64), lambda i: (0, 0)),
            pl.BlockSpec((1, 64), lambda i: (0, 0)),
        ],
        out_specs=pl.BlockSpec((ba, 64), lambda i: (i, 0)),
        out_shape=jax.ShapeDtypeStruct((NA, 64), F32),
    )(atom, w, b)


def _A_body(aa_ref, s0_ref, s1_ref, w1_ref, w2_ref, b_ref, o_ref):
    y = jnp.dot(aa_ref[...], w1_ref[...], preferred_element_type=F32)
    y += jnp.dot(s0_ref[...] + s1_ref[...], w2_ref[...],
                 preferred_element_type=F32)
    o_ref[...] = jnp.maximum(y + b_ref[...], 0.0)


def _A_call(aa, s0, s1, w1, w2, b):
    ba = 2000
    return pl.pallas_call(
        _A_body,
        grid=(NA // ba,),
        in_specs=[
            pl.BlockSpec((ba, 64), lambda i: (i, 0)),
            pl.BlockSpec((ba, 64), lambda i: (i, 0)),
            pl.BlockSpec((ba, 64), lambda i: (i, 0)),
            pl.BlockSpec((64, 64), lambda i: (0, 0)),
            pl.BlockSpec((64, 64), lambda i: (0, 0)),
            pl.BlockSpec((1, 64), lambda i: (0, 0)),
        ],
        out_specs=pl.BlockSpec((ba, 64), lambda i: (i, 0)),
        out_shape=jax.ShapeDtypeStruct((NA, 64), F32),
    )(aa, s0, s1, w1, w2, b)


def _P_fuse_body(ap_ref, pr_ref, wp1_ref, wp2_ref, wpp_ref, bpp_ref, bp_ref,
                 wpa_ref, bpa_ref, o_ref):
    pp = jnp.maximum(
        jnp.dot(pr_ref[...], wpp_ref[...], preferred_element_type=F32)
        + bpp_ref[...], 0.0)
    y = jnp.dot(ap_ref[...], wp1_ref[...], preferred_element_type=F32)
    y += jnp.dot(pp, wp2_ref[...], preferred_element_type=F32)
    p = jnp.maximum(y + bp_ref[...], 0.0)
    o_ref[...] = jnp.maximum(
        jnp.dot(p, wpa_ref[...], preferred_element_type=F32) + bpa_ref[...],
        0.0)


def _P_fuse_call(ap, pair, wp1, wp2, wpp, bpp, bp, wpa, bpa):
    pin = pair.shape[1]
    bpr = 4096
    return pl.pallas_call(
        _P_fuse_body,
        grid=(NPP // bpr,),
        in_specs=[
            pl.BlockSpec((bpr, 64), lambda i: (i, 0)),
            pl.BlockSpec((bpr, pin), lambda i: (i, 0)),
            pl.BlockSpec((64, 64), lambda i: (0, 0)),
            pl.BlockSpec((64, 64), lambda i: (0, 0)),
            pl.BlockSpec((pin, 64), lambda i: (0, 0)),
            pl.BlockSpec((1, 64), lambda i: (0, 0)),
            pl.BlockSpec((1, 64), lambda i: (0, 0)),
            pl.BlockSpec((64, 64), lambda i: (0, 0)),
            pl.BlockSpec((1, 64), lambda i: (0, 0)),
        ],
        out_specs=pl.BlockSpec((bpr, 64), lambda i: (i, 0)),
        out_shape=jax.ShapeDtypeStruct((NPP, 64), F32),
    )(ap, pair, wp1, wp2, wpp, bpp, bp, wpa, bpa)


def _head_body(a_ref, wd_ref, bd_ref, sc_ref, be_ref, wg_ref, o_ref, m_ref):
    h = jnp.tanh(
        jnp.dot(a_ref[...], wd_ref[...], preferred_element_type=F32)
        + bd_ref[...])
    h = h * sc_ref[...] + be_ref[...]
    den = jnp.zeros_like(h)
    for k in range(11):
        c = -0.5 / (_SIG[k] * _SIG[k])
        d = h - _MU[k]
        mk = jnp.exp(c * d * d)
        m_ref[k] = mk
        den = den + mk
    inv = 1.0 / den
    acc = jnp.zeros(o_ref.shape, F32)
    for k in range(11):
        acc = acc + jnp.dot(m_ref[k] * inv, wg_ref[k * 128:(k + 1) * 128, :],
                            preferred_element_type=F32)
    o_ref[...] = acc


def _head_call(a, wd, bd, scale, beta, wg):
    bh = 512
    return pl.pallas_call(
        _head_body,
        grid=(NAP // bh,),
        in_specs=[
            pl.BlockSpec((bh, 64), lambda i: (i, 0)),
            pl.BlockSpec((64, 128), lambda i: (0, 0)),
            pl.BlockSpec((1, 128), lambda i: (0, 0)),
            pl.BlockSpec((1, 128), lambda i: (0, 0)),
            pl.BlockSpec((1, 128), lambda i: (0, 0)),
            pl.BlockSpec((1408, 128), lambda i: (0, 0)),
        ],
        out_specs=pl.BlockSpec((bh, 128), lambda i: (i, 0)),
        out_shape=jax.ShapeDtypeStruct((NAP, 128), F32),
        scratch_shapes=[pltpu.VMEM((11, bh, 128), F32)],
    )(a, wd, bd, scale, beta, wg)


def _out_body(p0_ref, p1_ref, b_ref, o_ref):
    o_ref[...] = jnp.tanh(p0_ref[...] + p1_ref[...] + b_ref[...])


def _out_call(p0, p1, b):
    return pl.pallas_call(
        _out_body,
        grid=(1,),
        in_specs=[
            pl.BlockSpec((NM, 128), lambda i: (0, 0)),
            pl.BlockSpec((NM, 128), lambda i: (0, 0)),
            pl.BlockSpec((1, 128), lambda i: (0, 0)),
        ],
        out_specs=pl.BlockSpec((NM, 128), lambda i: (0, 0)),
        out_shape=jax.ShapeDtypeStruct((NM, 128), F32),
    )(p0, p1, b)


_MESH = plsc.VectorSubcoreMesh(core_axis_name="c", subcore_axis_name="s")


def _make_segsum(nrows, d, nseg, g, steps):
    rw = nrows // NW
    sr = rw // steps
    gs = sr // g
    gpw = gs * steps
    zr = nseg // 16
    assert sr % 8 == 0 and gpw % 8 == 0 and zr % 8 == 0 and gs * g == sr

    def body(vals_hbm, idx_hbm, zeros_hbm, out_hbm, acc_sh,
             vals0, vals1, idx_v, sem0, sem1):
        c = lax.axis_index("c")
        s = lax.axis_index("s")
        wid = s * 2 + c
        pltpu.sync_copy(zeros_hbm, acc_sh.at[pl.ds(s * zr, zr)])
        pltpu.sync_copy(idx_hbm.at[pl.ds(wid * gpw, gpw)], idx_v)
        plsc.subcore_barrier()

        bufs = ((vals0, sem0), (vals1, sem1))
        descs = [None, None]

        def fire(st):
            b = st % 2
            vals_v, sem = bufs[b]
            descs[b] = pltpu.async_copy(
                vals_hbm.at[pl.ds(wid * rw + st * sr, sr)], vals_v, sem)

        fire(0)
        for st in range(steps):
            b = st % 2
            if st + 1 < steps:
                fire(st + 1)
            descs[b].wait()
            vals_v, _ = bufs[b]
            for j in range(gs):
                pltpu.sync_copy(vals_v.at[pl.ds(j * g, g)],
                                acc_sh.at[idx_v.at[st * gs + j]], add=True)
        plsc.subcore_barrier()
        pltpu.sync_copy(acc_sh.at[pl.ds(s * zr, zr)],
                        out_hbm.at[pl.ds(c * nseg + s * zr, zr)])

    return pl.kernel(
        body,
        out_type=jax.ShapeDtypeStruct((2 * nseg, d), F32),
        mesh=_MESH,
        scratch_types=[
            pltpu.VMEM_SHARED((nseg, d), F32),
            pltpu.VMEM((sr, d), F32),
            pltpu.VMEM((sr, d), F32),
            pltpu.VMEM((gpw, g), jnp.int32),
            pltpu.SemaphoreType.DMA,
            pltpu.SemaphoreType.DMA,
        ])


def _make_gather():
    g = 128
    GT = NPP // NW // g * 2
    G_C0 = 40
    G_C1 = GT - G_C0
    GMAX = max(G_C0, G_C1)

    def body(x12_hbm, ii_hbm, jj_hbm, ap_hbm,
             gi0, gi1, gj0, gj1, ov0, ov1, ii_v, jj_v,
             semg0, semg1, semo0, semo1):
        c = lax.axis_index("c")
        s = lax.axis_index("s")
        base_g = s * GT + jnp.where(c == 0, 0, G_C0)
        ngrp = jnp.where(c == 0, G_C0, G_C1)
        pltpu.sync_copy(ii_hbm.at[pl.ds(base_g, GMAX)], ii_v)
        pltpu.sync_copy(jj_hbm.at[pl.ds(base_g, GMAX)], jj_v)

        gbufs = ((gi0, gj0, semg0), (gi1, gj1, semg1))
        obufs = ((ov0, semo0), (ov1, semo1))
        base = base_g * g

        def fire(grp, b):
            gi, gj, sem = gbufs[b]
            pltpu.async_copy(x12_hbm.at[ii_v.at[grp]], gi, sem)
            pltpu.async_copy(x12_hbm.at[jj_v.at[grp]], gj, sem)

        def wait_gather(b):
            gi, gj, sem = gbufs[b]
            pltpu.make_async_copy(x12_hbm.at[ii_v.at[0]], gi, sem).wait()
            pltpu.make_async_copy(x12_hbm.at[jj_v.at[0]], gj, sem).wait()

        def wait_out(b):
            ov, semo = obufs[b]
            pltpu.make_async_copy(ov, ap_hbm.at[pl.ds(0, g)], semo).wait()

        def process(grp, b, st):
            gi, gj, _ = gbufs[b]
            ov, semo = obufs[b]
            wait_gather(b)

            @pl.when(st > 0)
            def _():
                wait_out(b)

            def row(r, carry):
                for q in range(4):
                    x1i = gi[r, pl.ds(q * 16, 16)]
                    x2i = gi[r, pl.ds(64 + q * 16, 16)]
                    x1j = gj[r, pl.ds(q * 16, 16)]
                    x2j = gj[r, pl.ds(64 + q * 16, 16)]
                    t = (jnp.maximum(x1i + x2j, 0.0)
                         + jnp.maximum(x1j + x2i, 0.0))
                    ov[r, pl.ds(q * 16, 16)] = t
                return carry

            lax.fori_loop(0, g, row, 0, unroll=4)
            pltpu.async_copy(ov, ap_hbm.at[pl.ds(base + grp * g, g)], semo)

        nsteps = ngrp // 2
        fire(0, 0)

        def step(st, carry):
            g0 = st * 2
            fire(g0 + 1, 1)
            process(g0, 0, st)

            @pl.when(st < nsteps - 1)
            def _():
                fire(g0 + 2, 0)

            process(g0 + 1, 1, st)
            return carry

        lax.fori_loop(0, nsteps, step, 0)
        wait_out(0)
        wait_out(1)

    return pl.kernel(
        body,
        out_type=jax.ShapeDtypeStruct((NPP, 64), F32),
        mesh=_MESH,
        scratch_types=[
            pltpu.VMEM((g, 128), F32),
            pltpu.VMEM((g, 128), F32),
            pltpu.VMEM((g, 128), F32),
            pltpu.VMEM((g, 128), F32),
            pltpu.VMEM((g, 64), F32),
            pltpu.VMEM((g, 64), F32),
            pltpu.VMEM((GMAX, g), jnp.int32),
            pltpu.VMEM((GMAX, g), jnp.int32),
            pltpu.SemaphoreType.DMA,
            pltpu.SemaphoreType.DMA,
            pltpu.SemaphoreType.DMA,
            pltpu.SemaphoreType.DMA,
        ])


def kernel(atom_features, pair_features, params, pair_split, atom_split,
           atom_to_pair):
    pad_p = NPP - NP
    ps2 = jnp.concatenate([
        pair_split.astype(jnp.int32),
        jnp.full((pad_p,), NA, jnp.int32),
    ]).reshape(NPP // 128, 128)
    ii2 = jnp.concatenate([
        atom_to_pair[:, 0].astype(jnp.int32), jnp.zeros((pad_p,), jnp.int32),
    ]).reshape(NPP // 128, 128)
    jj2 = jnp.concatenate([
        atom_to_pair[:, 1].astype(jnp.int32), jnp.zeros((pad_p,), jnp.int32),
    ]).reshape(NPP // 128, 128)
    asp = jnp.concatenate([
        atom_split.astype(jnp.int32),
        jnp.full((NAP - NA,), NM, jnp.int32),
    ]).reshape(NAP // 40, 40)
    zeros_a = jnp.zeros((NSEG_A // 16, 64), F32)
    zeros_m = jnp.zeros((NSEG_M // 16, 128), F32)

    segsum_pairs = _make_segsum(NPP, 64, NSEG_A, g=128, steps=40)
    segsum_mols = _make_segsum(NAP, 128, NSEG_M, g=40, steps=1)
    gather_pairs = _make_gather()

    w_ap = params["W_AP0"]
    wcat = jnp.concatenate([w_ap[:128], w_ap[128:], params["W_AA0"]], axis=1)
    bcat = jnp.concatenate([
        jnp.zeros((64,), F32), params["b_AP0"],
        params["b_AA0"]]).reshape(1, 192)
    x12, aa0 = _atoms_call(atom_features, wcat, bcat)
    pa0 = _pa_call(pair_features, params["W_PA0"],
                   params["b_PA0"].reshape(1, 64))
    seg0 = segsum_pairs(pa0, ps2, zeros_a)
    wa = params["W_A0"]
    atom1 = _A_call(aa0, seg0[:NA], seg0[NSEG_A:NSEG_A + NA], wa[:64],
                    wa[64:], params["b_A0"].reshape(1, 64))
    ap0 = gather_pairs(x12, ii2, jj2)
    wp = params["W_P0"]
    pa1 = _P_fuse_call(ap0, pair_features, wp[:64], wp[64:], params["W_PP0"],
                       params["b_PP0"].reshape(1, 64),
                       params["b_P0"].reshape(1, 64),
                       params["W_PA1"], params["b_PA1"].reshape(1, 64))
    aa1 = _aa_call(atom1, params["W_AA1"], params["b_AA1"].reshape(1, 64))
    seg1 = segsum_pairs(pa1, ps2, zeros_a)
    wa = params["W_A1"]
    atom = _A_call(aa1, seg1[:NA], seg1[NSEG_A:NSEG_A + NA], wa[:64],
                   wa[64:], params["b_A1"].reshape(1, 64))

    scale = (params["bn_gamma"] / np.sqrt(1.0 + 1e-3)).reshape(1, 128)
    beta = params["bn_beta"].reshape(1, 128)
    wg = jnp.transpose(params["W_g"].reshape(128, 11, 128),
                       (1, 0, 2)).reshape(1408, 128)
    gpre = _head_call(atom, params["W_d"], params["b_d"].reshape(1, 128),
                      scale, beta, wg)
    msum = segsum_mols(gpre, asp, zeros_m)
    return _out_call(msum[:NM], msum[NSEG_M:NSEG_M + NM],
                     params["b_g"].reshape(1, 128))
